# baseline (speedup 1.0000x reference)
import functools

import jax
import jax.numpy as jnp
from jax import lax
from jax.experimental import pallas as pl
from jax.experimental.pallas import tpu as pltpu
from jax.experimental.pallas import tpu_sc as plsc

VOCAB = 1_000_000
D = 64
BATCH = 16384
HIST = 50
B_FLAT = BATCH * HIST

NC, NS, L = 2, 16, 16
NW = NC * NS
PER_W = B_FLAT // NW
IDXW = 128
CHUNK = 512
N_SUB = CHUNK // IDXW
N_CHUNKS = PER_W // CHUNK
SCALE = 8.0


def _emb_body(x_hbm, tab_hbm, out_hbm, idx_v, rows_v, sem):
    wid = lax.axis_index("s") * NC + lax.axis_index("c")
    idx_row0 = wid * (PER_W // IDXW)
    out_row0 = wid * PER_W

    def chunk_body(g, carry):
        pltpu.sync_copy(x_hbm.at[pl.ds(idx_row0 + g * N_SUB, N_SUB)], idx_v)
        cps = [
            pltpu.async_copy(
                tab_hbm.at[idx_v.at[j]],
                rows_v.at[pl.ds(j * IDXW, IDXW)],
                sem,
            )
            for j in range(N_SUB)
        ]
        for cp in cps:
            cp.wait()

        def scale_body(r, c2):
            for c in range(D // L):
                rows_v[r, pl.ds(c * L, L)] = rows_v[r, pl.ds(c * L, L)] * SCALE
            return c2

        lax.fori_loop(0, CHUNK, scale_body, 0)
        pltpu.sync_copy(rows_v, out_hbm.at[pl.ds(out_row0 + g * CHUNK, CHUNK)])
        return carry

    lax.fori_loop(0, N_CHUNKS, chunk_body, 0)


@functools.partial(
    pl.kernel,
    out_type=jax.ShapeDtypeStruct((B_FLAT, D), jnp.float32),
    mesh=plsc.VectorSubcoreMesh(core_axis_name="c", subcore_axis_name="s"),
    scratch_types=[
        pltpu.VMEM((N_SUB, IDXW), jnp.int32),
        pltpu.VMEM((CHUNK, D), jnp.float32),
        pltpu.SemaphoreType.DMA,
    ],
    compiler_params=pltpu.CompilerParams(use_tc_tiling_on_sc=False),
)
def _emb(x_hbm, tab_hbm, out_hbm, idx_v, rows_v, sem):
    _emb_body(x_hbm, tab_hbm, out_hbm, idx_v, rows_v, sem)


def kernel(x, embedding_table):
    x2 = x.reshape(B_FLAT // IDXW, IDXW)
    out = _emb(x2, embedding_table)
    return out.reshape(BATCH, HIST, D)


# speedup vs baseline: 1.0013x; 1.0013x over previous
import functools

import jax
import jax.numpy as jnp
from jax import lax
from jax.experimental import pallas as pl
from jax.experimental.pallas import tpu as pltpu
from jax.experimental.pallas import tpu_sc as plsc

VOCAB = 1_000_000
D = 64
BATCH = 16384
HIST = 50
B_FLAT = BATCH * HIST

NC, NS, L = 2, 16, 16
NW = NC * NS
PER_W = B_FLAT // NW
IDXW = 128
CHUNK = 512
N_SUB = CHUNK // IDXW
N_CHUNKS = PER_W // CHUNK
SCALE = 8.0


def _emb_body(x_hbm, tab_hbm, out_hbm, idx_v, rows_v, sem):
    wid = lax.axis_index("s") * NC + lax.axis_index("c")
    idx0 = wid * PER_W
    out_row0 = wid * PER_W

    def chunk_body(g, carry):
        pltpu.sync_copy(x_hbm.at[pl.ds(idx0 + g * CHUNK, CHUNK)], idx_v)
        cps = [
            pltpu.async_copy(
                tab_hbm.at[idx_v.at[pl.ds(j * IDXW, IDXW)]],
                rows_v.at[pl.ds(j * IDXW, IDXW)],
                sem,
            )
            for j in range(N_SUB)
        ]
        for cp in cps:
            cp.wait()

        def scale_body(r, c2):
            for c in range(D // L):
                rows_v[r, pl.ds(c * L, L)] = rows_v[r, pl.ds(c * L, L)] * SCALE
            return c2

        lax.fori_loop(0, CHUNK, scale_body, 0)
        pltpu.sync_copy(rows_v, out_hbm.at[pl.ds(out_row0 + g * CHUNK, CHUNK)])
        return carry

    lax.fori_loop(0, N_CHUNKS, chunk_body, 0)


@functools.partial(
    pl.kernel,
    out_type=jax.ShapeDtypeStruct((B_FLAT, D), jnp.float32),
    mesh=plsc.VectorSubcoreMesh(core_axis_name="c", subcore_axis_name="s"),
    scratch_types=[
        pltpu.VMEM((CHUNK,), jnp.int32),
        pltpu.VMEM((CHUNK, D), jnp.float32),
        pltpu.SemaphoreType.DMA,
    ],
    compiler_params=pltpu.CompilerParams(use_tc_tiling_on_sc=False),
)
def _emb(x_hbm, tab_hbm, out_hbm, idx_v, rows_v, sem):
    _emb_body(x_hbm, tab_hbm, out_hbm, idx_v, rows_v, sem)


def kernel(x, embedding_table):
    xf = x.reshape(B_FLAT)
    out = _emb(xf, embedding_table)
    return out.reshape(BATCH, HIST, D)
